# baseline (speedup 1.0000x reference)
import jax
import jax.numpy as jnp
from jax.experimental import pallas as pl
from jax.experimental.pallas import tpu as pltpu

N = 10000
E = 320000
H = 128


def _gat_layer(h, edge_d, src, dst, Wd, W1, W2, Wa):
    t = edge_d @ Wd
    z = h @ W1
    zi = h @ W2
    a = z @ Wa[:H]
    b = z @ Wa[H : 2 * H]
    e = jax.nn.leaky_relu(a[src] + b[dst] + t * Wa[2 * H], negative_slope=0.01)
    ee = jnp.exp(e)
    den = jax.ops.segment_sum(ee, dst, num_segments=N)
    alpha = ee / (den[dst] + 1e-9)
    zn = jax.ops.segment_sum(alpha * z[src], dst, num_segments=N)
    return jax.nn.relu(zi + zn)


def _bilinear_body(g_ref, hd_ref, bb_ref, out_ref):
    out_ref[...] = jnp.sum(g_ref[...] * hd_ref[...], axis=1, keepdims=True) + bb_ref[0]


def _bilinear(gs, hd, bb):
    B = 4000
    return pl.pallas_call(
        _bilinear_body,
        grid=(E // B,),
        in_specs=[
            pl.BlockSpec((B, H), lambda i: (i, 0)),
            pl.BlockSpec((B, H), lambda i: (i, 0)),
            pl.BlockSpec(memory_space=pltpu.SMEM),
        ],
        out_specs=pl.BlockSpec((B, 1), lambda i: (i, 0)),
        out_shape=jax.ShapeDtypeStruct((E, 1), jnp.float32),
    )(gs, hd, bb)


def kernel(x, edge_index, edge_d, Wd_o1, W1_o1, W2_o1, Wa_o1, Wd_o2, W1_o2, W2_o2, Wa_o2, Wd_d1, W1_d1, W2_d1, Wa_d1, Wd_d2, W1_d2, W2_d2, Wa_d2, Wb, bb):
    src = edge_index[0]
    dst = edge_index[1]
    h_o = _gat_layer(x, edge_d, src, dst, Wd_o1, W1_o1, W2_o1, Wa_o1)
    h_o = _gat_layer(h_o, edge_d, src, dst, Wd_o2, W1_o2, W2_o2, Wa_o2)
    h_d = _gat_layer(x, edge_d, src, dst, Wd_d1, W1_d1, W2_d1, Wa_d1)
    h_d = _gat_layer(h_d, edge_d, src, dst, Wd_d2, W1_d2, W2_d2, Wa_d2)
    gs = (h_o @ Wb[0])[src]
    hd = h_d[dst]
    return _bilinear(gs, hd, bb)


# speedup vs baseline: 9.2689x; 9.2689x over previous
import dataclasses
import functools

import jax
import jax.numpy as jnp
from jax import lax
from jax.experimental import pallas as pl
from jax.experimental.pallas import tpu as pltpu
from jax.experimental.pallas import tpu_sc as plsc

N = 10000
E = 320000
H = 128

NC = 2
NS = 16
NW = NC * NS
EPW = E // NW
K = 80
NCHUNK = EPW // K
ROUND = 2000
RCHUNK = ROUND // K
NPLANE = E // ROUND
ROWS_PER_TILE = 640

_SC_PARAMS = pltpu.CompilerParams()
if "needs_layout_passes" in pltpu.CompilerParams.__dataclass_fields__:
    _SC_PARAMS = dataclasses.replace(_SC_PARAMS, needs_layout_passes=False)

_MESH = plsc.VectorSubcoreMesh(core_axis_name="c", subcore_axis_name="s")

_PREC = jax.lax.Precision.HIGHEST


def _f32(*shape):
    return jax.ShapeDtypeStruct(shape, jnp.float32)


_BLK = 1000


def _dot(a, b):
    return jax.lax.dot_general(a, b, (((1,), (0,)), ((), ())),
                               precision=_PREC, preferred_element_type=jnp.float32)


def _proj_body(h_ref, w1_ref, w2_ref, z_ref, zi_ref):
    h = h_ref[...]
    z_ref[...] = _dot(h, w1_ref[...])
    zi_ref[...] = _dot(h, w2_ref[...])


def _tc_proj_first(h, W1, W2):
    return pl.pallas_call(
        _proj_body,
        grid=(N // _BLK,),
        in_specs=[
            pl.BlockSpec((_BLK, H), lambda i: (i, 0)),
            pl.BlockSpec((H, H), lambda i: (0, 0)),
            pl.BlockSpec((H, H), lambda i: (0, 0)),
        ],
        out_specs=[
            pl.BlockSpec((_BLK, H), lambda i: (i, 0)),
            pl.BlockSpec((_BLK, H), lambda i: (i, 0)),
        ],
        out_shape=[_f32(N, H), _f32(N, H)],
    )(h, W1, W2)


def _proj_mid_body(zi_prev_ref, zn_ref, w1_ref, w2_ref, z_ref, zi_ref):
    h = jnp.maximum(zi_prev_ref[...] + zn_ref[0] + zn_ref[1], 0.0)
    z_ref[...] = _dot(h, w1_ref[...])
    zi_ref[...] = _dot(h, w2_ref[...])


def _tc_proj_mid(zi_prev, zn_parts, W1, W2):
    return pl.pallas_call(
        _proj_mid_body,
        grid=(N // _BLK,),
        in_specs=[
            pl.BlockSpec((_BLK, H), lambda i: (i, 0)),
            pl.BlockSpec((2, _BLK, H), lambda i: (0, i, 0)),
            pl.BlockSpec((H, H), lambda i: (0, 0)),
            pl.BlockSpec((H, H), lambda i: (0, 0)),
        ],
        out_specs=[
            pl.BlockSpec((_BLK, H), lambda i: (i, 0)),
            pl.BlockSpec((_BLK, H), lambda i: (i, 0)),
        ],
        out_shape=[_f32(N, H), _f32(N, H)],
    )(zi_prev, zn_parts, W1, W2)


def _ab_body(z_ref, wab_ref, ab_ref):
    ab_ref[...] = jax.lax.dot_general(
        wab_ref[...], z_ref[...], (((0,), (1,)), ((), ())),
        precision=_PREC, preferred_element_type=jnp.float32)


def _tc_ab(z, Wab):
    return pl.pallas_call(
        _ab_body,
        in_specs=[
            pl.BlockSpec((N, H), lambda: (0, 0)),
            pl.BlockSpec((H, 8), lambda: (0, 0)),
        ],
        out_specs=pl.BlockSpec((8, N), lambda: (0, 0)),
        out_shape=_f32(8, N),
    )(z, Wab)


def _fin_body(zio_ref, zno_ref, zid_ref, znd_ref, wb_ref, g_ref, hd_ref):
    h_o = jnp.maximum(zio_ref[...] + zno_ref[0] + zno_ref[1], 0.0)
    g_ref[...] = _dot(h_o, wb_ref[...])
    hd_ref[...] = jnp.maximum(zid_ref[...] + znd_ref[0] + znd_ref[1], 0.0)


def _tc_final(zi_o, zn_o, zi_d, zn_d, Wb0):
    return pl.pallas_call(
        _fin_body,
        grid=(N // _BLK,),
        in_specs=[
            pl.BlockSpec((_BLK, H), lambda i: (i, 0)),
            pl.BlockSpec((2, _BLK, H), lambda i: (0, i, 0)),
            pl.BlockSpec((_BLK, H), lambda i: (i, 0)),
            pl.BlockSpec((2, _BLK, H), lambda i: (0, i, 0)),
            pl.BlockSpec((H, H), lambda i: (0, 0)),
        ],
        out_specs=[
            pl.BlockSpec((_BLK, H), lambda i: (i, 0)),
            pl.BlockSpec((_BLK, H), lambda i: (i, 0)),
        ],
        out_shape=[_f32(N, H), _f32(N, H)],
    )(zi_o, zn_o, zi_d, zn_d, Wb0)


def _leaky(x):
    return jnp.where(x >= 0.0, x, 0.01 * x)


@functools.partial(
    pl.kernel,
    out_type=_f32(NC, N, H),
    mesh=_MESH,
    compiler_params=_SC_PARAMS,
    scratch_types=[
        pltpu.VMEM((N,), jnp.float32),
        pltpu.VMEM((N,), jnp.float32),
        pltpu.VMEM((ROUND,), jnp.float32),
        pltpu.VMEM((ROUND,), jnp.int32),
        pltpu.VMEM((RCHUNK, K), jnp.int32),
        pltpu.VMEM((ROUND,), jnp.float32),
        pltpu.VMEM((K,), jnp.float32),
        pltpu.VMEM((K, H), jnp.float32),
        pltpu.VMEM((K,), jnp.float32),
        pltpu.VMEM((16,), jnp.float32),
        pltpu.VMEM_SHARED((N,), jnp.float32),
        pltpu.VMEM_SHARED((N, H), jnp.float32),
        pltpu.SemaphoreType.DMA,
    ],
)
def _sc_layer(z_hbm, ab_hbm, cvec_hbm, src_hbm, dst2_hbm, ed_hbm, out_hbm,
              a_v, b_v, ee_v, src_v, dst_v, ed_v, den_v, rows_v, al_v, c_v,
              den_sh, zn_sh, sem):
    c = lax.axis_index("c")
    s = lax.axis_index("s")
    w = c * NS + s

    pltpu.sync_copy(ab_hbm.at[0], a_v)
    pltpu.sync_copy(ab_hbm.at[1], b_v)
    pltpu.sync_copy(cvec_hbm, c_v)
    coeff = c_v[...]

    zero16 = jnp.zeros((16,), jnp.float32)

    @pl.loop(0, K)
    def _(r):
        for q in range(H // 16):
            rows_v[r, pl.ds(q * 16, 16)] = zero16

    nrows = jnp.where(s == NS - 1, N - (NS - 1) * ROWS_PER_TILE, ROWS_PER_TILE)
    base_row = s * ROWS_PER_TILE

    @pl.loop(0, nrows, step=K)
    def _(r0):
        pltpu.sync_copy(rows_v, zn_sh.at[pl.ds(base_row + r0, K)])

    @pl.when(s == 0)
    def _():
        @pl.loop(0, ROUND, step=16)
        def _(i):
            ee_v[pl.ds(i, 16)] = zero16

        @pl.loop(0, N, step=ROUND)
        def _(i):
            pltpu.sync_copy(ee_v, den_sh.at[pl.ds(i, ROUND)])

    plsc.subcore_barrier()

    @pl.loop(0, 2 * EPW // ROUND)
    def _(rr):
        base = s * (2 * EPW) + rr * ROUND
        plane = s * (2 * EPW // ROUND) + rr
        pltpu.sync_copy(src_hbm.at[pl.ds(base, ROUND)], src_v)
        pltpu.sync_copy(dst2_hbm.at[plane], dst_v)
        pltpu.sync_copy(ed_hbm.at[pl.ds(base, ROUND)], ed_v)

        @pl.loop(0, RCHUNK)
        def _(j):
            for g in range(K // 16):
                o16 = j * K + g * 16
                s16 = src_v[pl.ds(o16, 16)]
                d16 = dst_v[j, pl.ds(g * 16, 16)]
                av = plsc.load_gather(a_v, [s16])
                bv = plsc.load_gather(b_v, [d16])
                ed16 = ed_v[pl.ds(o16, 16)]
                ee_v[pl.ds(o16, 16)] = jnp.exp(_leaky(av + bv + coeff * ed16))

        @pl.loop(0, RCHUNK)
        def _(j):
            pltpu.async_copy(ee_v.at[pl.ds(j * K, K)], den_sh.at[dst_v.at[j]],
                             sem, add=True)

        @pl.loop(0, RCHUNK)
        def _(j):
            pltpu.make_async_copy(ee_v.at[pl.ds(0, K)],
                                  den_sh.at[dst_v.at[0]], sem).wait()

    plsc.subcore_barrier()

    @pl.loop(0, EPW // ROUND)
    def _(rr):
        base = w * EPW + rr * ROUND
        plane = w * (EPW // ROUND) + rr
        pltpu.sync_copy(src_hbm.at[pl.ds(base, ROUND)], src_v)
        pltpu.sync_copy(dst2_hbm.at[plane], dst_v)
        pltpu.sync_copy(ed_hbm.at[pl.ds(base, ROUND)], ed_v)

        @pl.loop(0, RCHUNK)
        def _(j):
            pltpu.sync_copy(z_hbm.at[src_v.at[pl.ds(j * K, K)]], rows_v)
            pltpu.sync_copy(den_sh.at[dst_v.at[j]], den_v)
            for g in range(K // 16):
                o16 = j * K + g * 16
                s16 = src_v[pl.ds(o16, 16)]
                d16 = dst_v[j, pl.ds(g * 16, 16)]
                av = plsc.load_gather(a_v, [s16])
                bv = plsc.load_gather(b_v, [d16])
                ed16 = ed_v[pl.ds(o16, 16)]
                ee16 = jnp.exp(_leaky(av + bv + coeff * ed16))
                dv = den_v[pl.ds(g * 16, 16)]
                al_v[pl.ds(g * 16, 16)] = ee16 / (dv + 1e-9)

            @pl.loop(0, K)
            def _(r):
                sc = plsc.load_gather(al_v, [jnp.broadcast_to(r, (16,))])
                for q in range(H // 16):
                    rows_v[r, pl.ds(q * 16, 16)] = (
                        rows_v[r, pl.ds(q * 16, 16)] * sc)

            pltpu.sync_copy(rows_v, zn_sh.at[dst_v.at[j]], add=True)

    plsc.subcore_barrier()

    @pl.loop(0, nrows, step=K)
    def _(r0):
        pltpu.sync_copy(zn_sh.at[pl.ds(base_row + r0, K)],
                        out_hbm.at[c, pl.ds(base_row + r0, K)])


@functools.partial(
    pl.kernel,
    out_type=_f32(E),
    mesh=_MESH,
    compiler_params=_SC_PARAMS,
    scratch_types=[
        pltpu.VMEM((EPW,), jnp.int32),
        pltpu.VMEM((EPW,), jnp.int32),
        pltpu.VMEM((K, H), jnp.float32),
        pltpu.VMEM((K, H), jnp.float32),
        pltpu.VMEM((EPW,), jnp.float32),
        pltpu.VMEM((16,), jnp.float32),
    ],
)
def _sc_bilinear(g_hbm, hd_hbm, src_hbm, dst_hbm, bvec_hbm, out_hbm,
                 src_v, dst_v, gr_v, hr_v, o_v, b_v):
    c = lax.axis_index("c")
    s = lax.axis_index("s")
    w = c * NS + s
    base = w * EPW

    pltpu.sync_copy(bvec_hbm, b_v)
    bias = b_v[...]
    pltpu.sync_copy(src_hbm.at[pl.ds(base, EPW)], src_v)
    pltpu.sync_copy(dst_hbm.at[pl.ds(base, EPW)], dst_v)

    lane = lax.iota(jnp.int32, 16)

    @pl.loop(0, NCHUNK)
    def _(j):
        pltpu.sync_copy(g_hbm.at[src_v.at[pl.ds(j * K, K)]], gr_v)
        pltpu.sync_copy(hd_hbm.at[dst_v.at[pl.ds(j * K, K)]], hr_v)

        @pl.loop(0, K, step=16)
        def _(r0):
            ridx = lane + r0
            acc = jnp.zeros((16,), jnp.float32)
            for h in range(H):
                gv = plsc.load_gather(gr_v, [ridx, jnp.broadcast_to(h, (16,))])
                hv = plsc.load_gather(hr_v, [ridx, jnp.broadcast_to(h, (16,))])
                acc = acc + gv * hv
            o_v[pl.ds(j * K + r0, 16)] = acc + bias

    pltpu.sync_copy(o_v, out_hbm.at[pl.ds(base, EPW)])


def _layer_weights(Wd, Wa):
    Wab = jnp.concatenate(
        [Wa[:H], Wa[H:2 * H], jnp.zeros((H, 6), jnp.float32)], axis=1)
    cvec = jnp.broadcast_to((Wd[0, 0] * Wa[2 * H, 0]).reshape(1), (16,))
    return Wab, cvec


def kernel(x, edge_index, edge_d, Wd_o1, W1_o1, W2_o1, Wa_o1, Wd_o2, W1_o2,
           W2_o2, Wa_o2, Wd_d1, W1_d1, W2_d1, Wa_d1, Wd_d2, W1_d2, W2_d2,
           Wa_d2, Wb, bb):
    src = edge_index[0]
    dst = edge_index[1]
    dst2 = dst.reshape(NPLANE, RCHUNK, K)
    ed = edge_d[:, 0]

    def layer_first(h, Wd, W1, W2, Wa):
        Wab, cvec = _layer_weights(Wd, Wa)
        z, zi = _tc_proj_first(h, W1, W2)
        ab = _tc_ab(z, Wab)
        zn = _sc_layer(z, ab[:2], cvec, src, dst2, ed)
        return zi, zn

    def layer_mid(zi_prev, zn_prev, Wd, W1, W2, Wa):
        Wab, cvec = _layer_weights(Wd, Wa)
        z, zi = _tc_proj_mid(zi_prev, zn_prev, W1, W2)
        ab = _tc_ab(z, Wab)
        zn = _sc_layer(z, ab[:2], cvec, src, dst2, ed)
        return zi, zn

    zi_o, zn_o = layer_first(x, Wd_o1, W1_o1, W2_o1, Wa_o1)
    zi_o, zn_o = layer_mid(zi_o, zn_o, Wd_o2, W1_o2, W2_o2, Wa_o2)
    zi_d, zn_d = layer_first(x, Wd_d1, W1_d1, W2_d1, Wa_d1)
    zi_d, zn_d = layer_mid(zi_d, zn_d, Wd_d2, W1_d2, W2_d2, Wa_d2)

    g, hd = _tc_final(zi_o, zn_o, zi_d, zn_d, Wb[0])
    bvec = jnp.broadcast_to(bb.reshape(1), (16,))
    out = _sc_bilinear(g, hd, src, dst, bvec)
    return out.reshape(E, 1)


# speedup vs baseline: 10.0793x; 1.0874x over previous
import dataclasses
import functools

import jax
import jax.numpy as jnp
from jax import lax
from jax.experimental import pallas as pl
from jax.experimental.pallas import tpu as pltpu
from jax.experimental.pallas import tpu_sc as plsc

N = 10000
E = 320000
H = 128

NC = 2
NS = 16
NW = NC * NS
EPW = E // NW
K = 80
NCHUNK = EPW // K
ROUND = 2000
RCHUNK = ROUND // K
NPLANE = E // ROUND
ROWS_PER_TILE = 640

_SC_PARAMS = pltpu.CompilerParams()
if "needs_layout_passes" in pltpu.CompilerParams.__dataclass_fields__:
    _SC_PARAMS = dataclasses.replace(_SC_PARAMS, needs_layout_passes=False)

_MESH = plsc.VectorSubcoreMesh(core_axis_name="c", subcore_axis_name="s")

_PREC = jax.lax.Precision.HIGHEST


def _f32(*shape):
    return jax.ShapeDtypeStruct(shape, jnp.float32)


_BLK = 1000


def _dot(a, b):
    return jax.lax.dot_general(a, b, (((1,), (0,)), ((), ())),
                               precision=_PREC, preferred_element_type=jnp.float32)


def _proj_body(h_ref, w1_ref, w2_ref, z_ref, zi_ref):
    h = h_ref[...]
    z_ref[...] = _dot(h, w1_ref[...])
    zi_ref[...] = _dot(h, w2_ref[...])


def _tc_proj_first(h, W1, W2):
    return pl.pallas_call(
        _proj_body,
        grid=(N // _BLK,),
        in_specs=[
            pl.BlockSpec((_BLK, H), lambda i: (i, 0)),
            pl.BlockSpec((H, H), lambda i: (0, 0)),
            pl.BlockSpec((H, H), lambda i: (0, 0)),
        ],
        out_specs=[
            pl.BlockSpec((_BLK, H), lambda i: (i, 0)),
            pl.BlockSpec((_BLK, H), lambda i: (i, 0)),
        ],
        out_shape=[_f32(N, H), _f32(N, H)],
    )(h, W1, W2)


def _proj_mid_body(zi_prev_ref, zn_ref, w1_ref, w2_ref, z_ref, zi_ref):
    h = jnp.maximum(zi_prev_ref[...] + zn_ref[0] + zn_ref[1], 0.0)
    z_ref[...] = _dot(h, w1_ref[...])
    zi_ref[...] = _dot(h, w2_ref[...])


def _tc_proj_mid(zi_prev, zn_parts, W1, W2):
    return pl.pallas_call(
        _proj_mid_body,
        grid=(N // _BLK,),
        in_specs=[
            pl.BlockSpec((_BLK, H), lambda i: (i, 0)),
            pl.BlockSpec((2, _BLK, H), lambda i: (0, i, 0)),
            pl.BlockSpec((H, H), lambda i: (0, 0)),
            pl.BlockSpec((H, H), lambda i: (0, 0)),
        ],
        out_specs=[
            pl.BlockSpec((_BLK, H), lambda i: (i, 0)),
            pl.BlockSpec((_BLK, H), lambda i: (i, 0)),
        ],
        out_shape=[_f32(N, H), _f32(N, H)],
    )(zi_prev, zn_parts, W1, W2)


def _ab_body(z_ref, wab_ref, ab_ref):
    ab_ref[...] = jax.lax.dot_general(
        wab_ref[...], z_ref[...], (((0,), (1,)), ((), ())),
        precision=_PREC, preferred_element_type=jnp.float32)


def _tc_ab(z, Wab):
    return pl.pallas_call(
        _ab_body,
        in_specs=[
            pl.BlockSpec((N, H), lambda: (0, 0)),
            pl.BlockSpec((H, 8), lambda: (0, 0)),
        ],
        out_specs=pl.BlockSpec((8, N), lambda: (0, 0)),
        out_shape=_f32(8, N),
    )(z, Wab)


def _fin_body(zio_ref, zno_ref, zid_ref, znd_ref, wb_ref, g_ref, hd_ref):
    h_o = jnp.maximum(zio_ref[...] + zno_ref[0] + zno_ref[1], 0.0)
    g_ref[...] = _dot(h_o, wb_ref[...])
    hd_ref[...] = jnp.maximum(zid_ref[...] + znd_ref[0] + znd_ref[1], 0.0)


def _tc_final(zi_o, zn_o, zi_d, zn_d, Wb0):
    return pl.pallas_call(
        _fin_body,
        grid=(N // _BLK,),
        in_specs=[
            pl.BlockSpec((_BLK, H), lambda i: (i, 0)),
            pl.BlockSpec((2, _BLK, H), lambda i: (0, i, 0)),
            pl.BlockSpec((_BLK, H), lambda i: (i, 0)),
            pl.BlockSpec((2, _BLK, H), lambda i: (0, i, 0)),
            pl.BlockSpec((H, H), lambda i: (0, 0)),
        ],
        out_specs=[
            pl.BlockSpec((_BLK, H), lambda i: (i, 0)),
            pl.BlockSpec((_BLK, H), lambda i: (i, 0)),
        ],
        out_shape=[_f32(N, H), _f32(N, H)],
    )(zi_o, zn_o, zi_d, zn_d, Wb0)


def _leaky(x):
    return jnp.where(x >= 0.0, x, 0.01 * x)


@functools.partial(
    pl.kernel,
    out_type=_f32(NC, N, H),
    mesh=_MESH,
    compiler_params=_SC_PARAMS,
    scratch_types=[
        pltpu.VMEM((N,), jnp.float32),
        pltpu.VMEM((N,), jnp.float32),
        pltpu.VMEM((ROUND,), jnp.float32),
        pltpu.VMEM((ROUND,), jnp.int32),
        pltpu.VMEM((RCHUNK, K), jnp.int32),
        pltpu.VMEM((ROUND,), jnp.float32),
        pltpu.VMEM((K,), jnp.float32),
        pltpu.VMEM((K, H), jnp.float32),
        pltpu.VMEM((K,), jnp.float32),
        pltpu.VMEM((16,), jnp.float32),
        pltpu.VMEM_SHARED((N,), jnp.float32),
        pltpu.VMEM_SHARED((N, H), jnp.float32),
        pltpu.SemaphoreType.DMA,
    ],
)
def _sc_layer(z_hbm, ab_hbm, cvec_hbm, src_hbm, dst2_hbm, ed_hbm, out_hbm,
              a_v, b_v, ee_v, src_v, dst_v, ed_v, den_v, rows_v, al_v, c_v,
              den_sh, zn_sh, sem):
    c = lax.axis_index("c")
    s = lax.axis_index("s")
    w = c * NS + s

    pltpu.sync_copy(ab_hbm.at[0], a_v)
    pltpu.sync_copy(ab_hbm.at[1], b_v)
    pltpu.sync_copy(cvec_hbm, c_v)
    coeff = c_v[...]

    zero16 = jnp.zeros((16,), jnp.float32)

    @pl.loop(0, K)
    def _(r):
        for q in range(H // 16):
            rows_v[r, pl.ds(q * 16, 16)] = zero16

    nrows = jnp.where(s == NS - 1, N - (NS - 1) * ROWS_PER_TILE, ROWS_PER_TILE)
    base_row = s * ROWS_PER_TILE

    @pl.loop(0, nrows, step=K)
    def _(r0):
        pltpu.sync_copy(rows_v, zn_sh.at[pl.ds(base_row + r0, K)])

    @pl.when(s == 0)
    def _():
        @pl.loop(0, ROUND, step=16)
        def _(i):
            ee_v[pl.ds(i, 16)] = zero16

        @pl.loop(0, N, step=ROUND)
        def _(i):
            pltpu.sync_copy(ee_v, den_sh.at[pl.ds(i, ROUND)])

    plsc.subcore_barrier()

    @pl.loop(0, 2 * EPW // ROUND)
    def _(rr):
        base = s * (2 * EPW) + rr * ROUND
        plane = s * (2 * EPW // ROUND) + rr
        pltpu.sync_copy(src_hbm.at[pl.ds(base, ROUND)], src_v)
        pltpu.sync_copy(dst2_hbm.at[plane], dst_v)
        pltpu.sync_copy(ed_hbm.at[pl.ds(base, ROUND)], ed_v)

        @pl.loop(0, RCHUNK)
        def _(j):
            for g in range(K // 16):
                o16 = j * K + g * 16
                s16 = src_v[pl.ds(o16, 16)]
                d16 = dst_v[j, pl.ds(g * 16, 16)]
                av = plsc.load_gather(a_v, [s16])
                bv = plsc.load_gather(b_v, [d16])
                ed16 = ed_v[pl.ds(o16, 16)]
                ee_v[pl.ds(o16, 16)] = jnp.exp(_leaky(av + bv + coeff * ed16))

        @pl.loop(0, RCHUNK)
        def _(j):
            pltpu.async_copy(ee_v.at[pl.ds(j * K, K)], den_sh.at[dst_v.at[j]],
                             sem, add=True)

        @pl.loop(0, RCHUNK)
        def _(j):
            pltpu.make_async_copy(ee_v.at[pl.ds(0, K)],
                                  den_sh.at[dst_v.at[0]], sem).wait()

    plsc.subcore_barrier()

    @pl.loop(0, EPW // ROUND)
    def _(rr):
        base = w * EPW + rr * ROUND
        plane = w * (EPW // ROUND) + rr
        pltpu.sync_copy(src_hbm.at[pl.ds(base, ROUND)], src_v)
        pltpu.sync_copy(dst2_hbm.at[plane], dst_v)
        pltpu.sync_copy(ed_hbm.at[pl.ds(base, ROUND)], ed_v)

        @pl.loop(0, RCHUNK)
        def _(j):
            pltpu.sync_copy(z_hbm.at[src_v.at[pl.ds(j * K, K)]], rows_v)
            pltpu.sync_copy(den_sh.at[dst_v.at[j]], den_v)
            for g in range(K // 16):
                o16 = j * K + g * 16
                s16 = src_v[pl.ds(o16, 16)]
                d16 = dst_v[j, pl.ds(g * 16, 16)]
                av = plsc.load_gather(a_v, [s16])
                bv = plsc.load_gather(b_v, [d16])
                ed16 = ed_v[pl.ds(o16, 16)]
                ee16 = jnp.exp(_leaky(av + bv + coeff * ed16))
                dv = den_v[pl.ds(g * 16, 16)]
                al_v[pl.ds(g * 16, 16)] = ee16 / (dv + 1e-9)

            @pl.loop(0, K)
            def _(r):
                sc = plsc.load_gather(al_v, [jnp.broadcast_to(r, (16,))])
                for q in range(H // 16):
                    rows_v[r, pl.ds(q * 16, 16)] = (
                        rows_v[r, pl.ds(q * 16, 16)] * sc)

            pltpu.sync_copy(rows_v, zn_sh.at[dst_v.at[j]], add=True)

    plsc.subcore_barrier()

    @pl.loop(0, nrows, step=K)
    def _(r0):
        pltpu.sync_copy(zn_sh.at[pl.ds(base_row + r0, K)],
                        out_hbm.at[c, pl.ds(base_row + r0, K)])


@functools.partial(
    pl.kernel,
    out_type=_f32(E),
    mesh=_MESH,
    compiler_params=_SC_PARAMS,
    scratch_types=[
        pltpu.VMEM((EPW,), jnp.int32),
        pltpu.VMEM((EPW,), jnp.int32),
        pltpu.VMEM((K, H), jnp.float32),
        pltpu.VMEM((K, H), jnp.float32),
        pltpu.VMEM((K, H), jnp.float32),
        pltpu.VMEM((K, H), jnp.float32),
        pltpu.VMEM((EPW,), jnp.float32),
        pltpu.VMEM((16,), jnp.float32),
        pltpu.SemaphoreType.DMA,
        pltpu.SemaphoreType.DMA,
    ],
)
def _sc_bilinear(g_hbm, hd_hbm, src_hbm, dst_hbm, bvec_hbm, out_hbm,
                 src_v, dst_v, gr0, hr0, gr1, hr1, o_v, b_v, semg0, semg1):
    c = lax.axis_index("c")
    s = lax.axis_index("s")
    w = c * NS + s
    base = w * EPW

    pltpu.sync_copy(bvec_hbm, b_v)
    bias = b_v[...]
    pltpu.sync_copy(src_hbm.at[pl.ds(base, EPW)], src_v)
    pltpu.sync_copy(dst_hbm.at[pl.ds(base, EPW)], dst_v)

    lane = lax.iota(jnp.int32, 16)

    def b_start(j, gr, hr, semg):
        pltpu.async_copy(g_hbm.at[src_v.at[pl.ds(j * K, K)]], gr, semg)
        pltpu.async_copy(hd_hbm.at[dst_v.at[pl.ds(j * K, K)]], hr, semg)

    def b_wait(j, gr, hr, semg):
        pltpu.make_async_copy(g_hbm.at[src_v.at[pl.ds(j * K, K)]], gr,
                              semg).wait()
        pltpu.make_async_copy(hd_hbm.at[dst_v.at[pl.ds(j * K, K)]], hr,
                              semg).wait()

    def b_compute(j, gr, hr):
        @pl.loop(0, K, step=16)
        def _(r0):
            ridx = lane + r0
            acc = jnp.zeros((16,), jnp.float32)
            for h in range(H):
                gv = plsc.load_gather(gr, [ridx, jnp.broadcast_to(h, (16,))])
                hv = plsc.load_gather(hr, [ridx, jnp.broadcast_to(h, (16,))])
                acc = acc + gv * hv
            o_v[pl.ds(j * K + r0, 16)] = acc + bias

    b_start(0, gr0, hr0, semg0)

    @pl.loop(0, NCHUNK - 1, step=2)
    def _(j):
        b_wait(j, gr0, hr0, semg0)
        b_start(j + 1, gr1, hr1, semg1)
        b_compute(j, gr0, hr0)
        b_wait(j + 1, gr1, hr1, semg1)
        b_start(j + 2, gr0, hr0, semg0)
        b_compute(j + 1, gr1, hr1)

    b_wait(NCHUNK - 1, gr0, hr0, semg0)
    b_compute(NCHUNK - 1, gr0, hr0)

    pltpu.sync_copy(o_v, out_hbm.at[pl.ds(base, EPW)])


def _layer_weights(Wd, Wa):
    Wab = jnp.concatenate(
        [Wa[:H], Wa[H:2 * H], jnp.zeros((H, 6), jnp.float32)], axis=1)
    cvec = jnp.broadcast_to((Wd[0, 0] * Wa[2 * H, 0]).reshape(1), (16,))
    return Wab, cvec


def kernel(x, edge_index, edge_d, Wd_o1, W1_o1, W2_o1, Wa_o1, Wd_o2, W1_o2,
           W2_o2, Wa_o2, Wd_d1, W1_d1, W2_d1, Wa_d1, Wd_d2, W1_d2, W2_d2,
           Wa_d2, Wb, bb):
    src = edge_index[0]
    dst = edge_index[1]
    dst2 = dst.reshape(NPLANE, RCHUNK, K)
    ed = edge_d[:, 0]

    def layer_first(h, Wd, W1, W2, Wa):
        Wab, cvec = _layer_weights(Wd, Wa)
        z, zi = _tc_proj_first(h, W1, W2)
        ab = _tc_ab(z, Wab)
        zn = _sc_layer(z, ab[:2], cvec, src, dst2, ed)
        return zi, zn

    def layer_mid(zi_prev, zn_prev, Wd, W1, W2, Wa):
        Wab, cvec = _layer_weights(Wd, Wa)
        z, zi = _tc_proj_mid(zi_prev, zn_prev, W1, W2)
        ab = _tc_ab(z, Wab)
        zn = _sc_layer(z, ab[:2], cvec, src, dst2, ed)
        return zi, zn

    zi_o, zn_o = layer_first(x, Wd_o1, W1_o1, W2_o1, Wa_o1)
    zi_o, zn_o = layer_mid(zi_o, zn_o, Wd_o2, W1_o2, W2_o2, Wa_o2)
    zi_d, zn_d = layer_first(x, Wd_d1, W1_d1, W2_d1, Wa_d1)
    zi_d, zn_d = layer_mid(zi_d, zn_d, Wd_d2, W1_d2, W2_d2, Wa_d2)

    g, hd = _tc_final(zi_o, zn_o, zi_d, zn_d, Wb[0])
    bvec = jnp.broadcast_to(bb.reshape(1), (16,))
    out = _sc_bilinear(g, hd, src, dst, bvec)
    return out.reshape(E, 1)


# speedup vs baseline: 15.3400x; 1.5219x over previous
import dataclasses
import functools

import jax
import jax.numpy as jnp
from jax import lax
from jax.experimental import pallas as pl
from jax.experimental.pallas import tpu as pltpu
from jax.experimental.pallas import tpu_sc as plsc

N = 10000
E = 320000
H = 128

NC = 2
NS = 16
NW = NC * NS
EPW = E // NW
K = 80
NCHUNK = EPW // K
ROUND = 2000
RCHUNK = ROUND // K
NPLANE = E // ROUND
ROWS_PER_TILE = 640

_SC_PARAMS = pltpu.CompilerParams()
if "needs_layout_passes" in pltpu.CompilerParams.__dataclass_fields__:
    _SC_PARAMS = dataclasses.replace(_SC_PARAMS, needs_layout_passes=False)

_MESH = plsc.VectorSubcoreMesh(core_axis_name="c", subcore_axis_name="s")

_PREC = jax.lax.Precision.HIGHEST


def _f32(*shape):
    return jax.ShapeDtypeStruct(shape, jnp.float32)


_BLK = 1000


def _dot(a, b):
    return jax.lax.dot_general(a, b, (((1,), (0,)), ((), ())),
                               precision=_PREC, preferred_element_type=jnp.float32)


def _proj_body(h_ref, w1_ref, w2_ref, z_ref, zi_ref):
    h = h_ref[...]
    z_ref[...] = _dot(h, w1_ref[...])
    zi_ref[...] = _dot(h, w2_ref[...])


def _tc_proj_first(h, W1, W2):
    return pl.pallas_call(
        _proj_body,
        grid=(N // _BLK,),
        in_specs=[
            pl.BlockSpec((_BLK, H), lambda i: (i, 0)),
            pl.BlockSpec((H, H), lambda i: (0, 0)),
            pl.BlockSpec((H, H), lambda i: (0, 0)),
        ],
        out_specs=[
            pl.BlockSpec((_BLK, H), lambda i: (i, 0)),
            pl.BlockSpec((_BLK, H), lambda i: (i, 0)),
        ],
        out_shape=[_f32(N, H), _f32(N, H)],
    )(h, W1, W2)


def _proj_mid_body(zi_prev_ref, zn_ref, w1_ref, w2_ref, z_ref, zi_ref):
    h = jnp.maximum(zi_prev_ref[...] + zn_ref[0] + zn_ref[1], 0.0)
    z_ref[...] = _dot(h, w1_ref[...])
    zi_ref[...] = _dot(h, w2_ref[...])


def _tc_proj_mid(zi_prev, zn_parts, W1, W2):
    return pl.pallas_call(
        _proj_mid_body,
        grid=(N // _BLK,),
        in_specs=[
            pl.BlockSpec((_BLK, H), lambda i: (i, 0)),
            pl.BlockSpec((2, _BLK, H), lambda i: (0, i, 0)),
            pl.BlockSpec((H, H), lambda i: (0, 0)),
            pl.BlockSpec((H, H), lambda i: (0, 0)),
        ],
        out_specs=[
            pl.BlockSpec((_BLK, H), lambda i: (i, 0)),
            pl.BlockSpec((_BLK, H), lambda i: (i, 0)),
        ],
        out_shape=[_f32(N, H), _f32(N, H)],
    )(zi_prev, zn_parts, W1, W2)


def _ab_body(z_ref, wab_ref, ab_ref):
    ab_ref[...] = jax.lax.dot_general(
        wab_ref[...], z_ref[...], (((0,), (1,)), ((), ())),
        precision=_PREC, preferred_element_type=jnp.float32)


def _tc_ab(z, Wab):
    return pl.pallas_call(
        _ab_body,
        in_specs=[
            pl.BlockSpec((N, H), lambda: (0, 0)),
            pl.BlockSpec((H, 8), lambda: (0, 0)),
        ],
        out_specs=pl.BlockSpec((8, N), lambda: (0, 0)),
        out_shape=_f32(8, N),
    )(z, Wab)


def _fin_body(zio_ref, zno_ref, zid_ref, znd_ref, wb_ref, g_ref, hd_ref):
    h_o = jnp.maximum(zio_ref[...] + zno_ref[0] + zno_ref[1], 0.0)
    g_ref[...] = _dot(h_o, wb_ref[...])
    hd_ref[...] = jnp.maximum(zid_ref[...] + znd_ref[0] + znd_ref[1], 0.0)


def _tc_final(zi_o, zn_o, zi_d, zn_d, Wb0):
    return pl.pallas_call(
        _fin_body,
        grid=(N // _BLK,),
        in_specs=[
            pl.BlockSpec((_BLK, H), lambda i: (i, 0)),
            pl.BlockSpec((2, _BLK, H), lambda i: (0, i, 0)),
            pl.BlockSpec((_BLK, H), lambda i: (i, 0)),
            pl.BlockSpec((2, _BLK, H), lambda i: (0, i, 0)),
            pl.BlockSpec((H, H), lambda i: (0, 0)),
        ],
        out_specs=[
            pl.BlockSpec((_BLK, H), lambda i: (i, 0)),
            pl.BlockSpec((_BLK, H), lambda i: (i, 0)),
        ],
        out_shape=[_f32(N, H), _f32(N, H)],
    )(zi_o, zn_o, zi_d, zn_d, Wb0)


def _leaky(x):
    return jnp.where(x >= 0.0, x, 0.01 * x)


@functools.partial(
    pl.kernel,
    out_type=_f32(NC, N, H),
    mesh=_MESH,
    compiler_params=_SC_PARAMS,
    scratch_types=[
        pltpu.VMEM((N,), jnp.float32),
        pltpu.VMEM((N,), jnp.float32),
        pltpu.VMEM((ROUND,), jnp.float32),
        pltpu.VMEM((ROUND,), jnp.int32),
        pltpu.VMEM((RCHUNK, K), jnp.int32),
        pltpu.VMEM((ROUND,), jnp.float32),
        pltpu.VMEM((K,), jnp.float32),
        pltpu.VMEM((K, H), jnp.float32),
        pltpu.VMEM((K,), jnp.float32),
        pltpu.VMEM((16,), jnp.float32),
        pltpu.VMEM_SHARED((N,), jnp.float32),
        pltpu.VMEM_SHARED((N, H), jnp.float32),
        pltpu.SemaphoreType.DMA,
    ],
)
def _sc_layer(z_hbm, ab_hbm, cvec_hbm, src_hbm, dst2_hbm, ed_hbm, out_hbm,
              a_v, b_v, ee_v, src_v, dst_v, ed_v, den_v, rows_v, al_v, c_v,
              den_sh, zn_sh, sem):
    c = lax.axis_index("c")
    s = lax.axis_index("s")
    w = c * NS + s

    pltpu.sync_copy(ab_hbm.at[0], a_v)
    pltpu.sync_copy(ab_hbm.at[1], b_v)
    pltpu.sync_copy(cvec_hbm, c_v)
    coeff = c_v[...]

    zero16 = jnp.zeros((16,), jnp.float32)

    @pl.loop(0, K)
    def _(r):
        for q in range(H // 16):
            rows_v[r, pl.ds(q * 16, 16)] = zero16

    nrows = jnp.where(s == NS - 1, N - (NS - 1) * ROWS_PER_TILE, ROWS_PER_TILE)
    base_row = s * ROWS_PER_TILE

    @pl.loop(0, nrows, step=K)
    def _(r0):
        pltpu.sync_copy(rows_v, zn_sh.at[pl.ds(base_row + r0, K)])

    @pl.when(s == 0)
    def _():
        @pl.loop(0, ROUND, step=16)
        def _(i):
            ee_v[pl.ds(i, 16)] = zero16

        @pl.loop(0, N, step=ROUND)
        def _(i):
            pltpu.sync_copy(ee_v, den_sh.at[pl.ds(i, ROUND)])

    plsc.subcore_barrier()

    @pl.loop(0, 2 * EPW // ROUND)
    def _(rr):
        base = s * (2 * EPW) + rr * ROUND
        plane = s * (2 * EPW // ROUND) + rr
        pltpu.sync_copy(src_hbm.at[pl.ds(base, ROUND)], src_v)
        pltpu.sync_copy(dst2_hbm.at[plane], dst_v)
        pltpu.sync_copy(ed_hbm.at[pl.ds(base, ROUND)], ed_v)

        @pl.loop(0, RCHUNK)
        def _(j):
            for g in range(K // 16):
                o16 = j * K + g * 16
                s16 = src_v[pl.ds(o16, 16)]
                d16 = dst_v[j, pl.ds(g * 16, 16)]
                av = plsc.load_gather(a_v, [s16])
                bv = plsc.load_gather(b_v, [d16])
                ed16 = ed_v[pl.ds(o16, 16)]
                ee_v[pl.ds(o16, 16)] = jnp.exp(_leaky(av + bv + coeff * ed16))

        @pl.loop(0, RCHUNK)
        def _(j):
            pltpu.async_copy(ee_v.at[pl.ds(j * K, K)], den_sh.at[dst_v.at[j]],
                             sem, add=True)

        @pl.loop(0, RCHUNK)
        def _(j):
            pltpu.make_async_copy(ee_v.at[pl.ds(0, K)],
                                  den_sh.at[dst_v.at[0]], sem).wait()

    plsc.subcore_barrier()

    @pl.loop(0, EPW // ROUND)
    def _(rr):
        base = w * EPW + rr * ROUND
        plane = w * (EPW // ROUND) + rr
        pltpu.sync_copy(src_hbm.at[pl.ds(base, ROUND)], src_v)
        pltpu.sync_copy(dst2_hbm.at[plane], dst_v)
        pltpu.sync_copy(ed_hbm.at[pl.ds(base, ROUND)], ed_v)

        @pl.loop(0, RCHUNK)
        def _(j):
            pltpu.sync_copy(z_hbm.at[src_v.at[pl.ds(j * K, K)]], rows_v)
            pltpu.sync_copy(den_sh.at[dst_v.at[j]], den_v)
            for g in range(K // 16):
                o16 = j * K + g * 16
                s16 = src_v[pl.ds(o16, 16)]
                d16 = dst_v[j, pl.ds(g * 16, 16)]
                av = plsc.load_gather(a_v, [s16])
                bv = plsc.load_gather(b_v, [d16])
                ed16 = ed_v[pl.ds(o16, 16)]
                ee16 = jnp.exp(_leaky(av + bv + coeff * ed16))
                dv = den_v[pl.ds(g * 16, 16)]
                al_v[pl.ds(g * 16, 16)] = ee16 / (dv + 1e-9)

            @pl.loop(0, K)
            def _(r):
                sc = plsc.load_gather(al_v, [jnp.broadcast_to(r, (16,))])
                for q in range(H // 16):
                    rows_v[r, pl.ds(q * 16, 16)] = (
                        rows_v[r, pl.ds(q * 16, 16)] * sc)

            pltpu.sync_copy(rows_v, zn_sh.at[dst_v.at[j]], add=True)

    plsc.subcore_barrier()

    @pl.loop(0, nrows, step=K)
    def _(r0):
        pltpu.sync_copy(zn_sh.at[pl.ds(base_row + r0, K)],
                        out_hbm.at[c, pl.ds(base_row + r0, K)])


@functools.partial(
    pl.kernel,
    out_type=_f32(E),
    mesh=_MESH,
    compiler_params=_SC_PARAMS,
    scratch_types=[
        pltpu.VMEM((EPW,), jnp.int32),
        pltpu.VMEM((EPW,), jnp.int32),
        pltpu.VMEM((K, H), jnp.float32),
        pltpu.VMEM((K, H), jnp.float32),
        pltpu.VMEM((K, H), jnp.float32),
        pltpu.VMEM((K, H), jnp.float32),
        pltpu.VMEM((EPW,), jnp.float32),
        pltpu.VMEM((16,), jnp.float32),
        pltpu.VMEM((16, 16), jnp.float32),
        pltpu.SemaphoreType.DMA,
        pltpu.SemaphoreType.DMA,
    ],
)
def _sc_bilinear(g_hbm, hd_hbm, src_hbm, dst_hbm, bvec_hbm, out_hbm,
                 src_v, dst_v, gr0, hr0, gr1, hr1, o_v, b_v, t_v,
                 semg0, semg1):
    c = lax.axis_index("c")
    s = lax.axis_index("s")
    w = c * NS + s
    base = w * EPW

    pltpu.sync_copy(bvec_hbm, b_v)
    bias = b_v[...]
    pltpu.sync_copy(src_hbm.at[pl.ds(base, EPW)], src_v)
    pltpu.sync_copy(dst_hbm.at[pl.ds(base, EPW)], dst_v)

    lane = lax.iota(jnp.int32, 16)

    def b_start(j, gr, hr, semg):
        pltpu.async_copy(g_hbm.at[src_v.at[pl.ds(j * K, K)]], gr, semg)
        pltpu.async_copy(hd_hbm.at[dst_v.at[pl.ds(j * K, K)]], hr, semg)

    def b_wait(j, gr, hr, semg):
        pltpu.make_async_copy(g_hbm.at[src_v.at[pl.ds(j * K, K)]], gr,
                              semg).wait()
        pltpu.make_async_copy(hd_hbm.at[dst_v.at[pl.ds(j * K, K)]], hr,
                              semg).wait()

    def b_compute(j, gr, hr):
        @pl.loop(0, K, step=16)
        def _(r0):
            @pl.loop(0, 16)
            def _(rr):
                r = r0 + rr
                acc = gr[r, pl.ds(0, 16)] * hr[r, pl.ds(0, 16)]
                for q in range(1, H // 16):
                    acc = acc + gr[r, pl.ds(q * 16, 16)] * hr[r, pl.ds(q * 16, 16)]
                t_v[rr, pl.ds(0, 16)] = acc

            tot = bias
            for ell in range(16):
                tot = tot + plsc.load_gather(
                    t_v, [lane, jnp.broadcast_to(ell, (16,))])
            o_v[pl.ds(j * K + r0, 16)] = tot

    b_start(0, gr0, hr0, semg0)

    @pl.loop(0, NCHUNK - 1, step=2)
    def _(j):
        b_wait(j, gr0, hr0, semg0)
        b_start(j + 1, gr1, hr1, semg1)
        b_compute(j, gr0, hr0)
        b_wait(j + 1, gr1, hr1, semg1)
        b_start(j + 2, gr0, hr0, semg0)
        b_compute(j + 1, gr1, hr1)

    b_wait(NCHUNK - 1, gr0, hr0, semg0)
    b_compute(NCHUNK - 1, gr0, hr0)

    pltpu.sync_copy(o_v, out_hbm.at[pl.ds(base, EPW)])


def _layer_weights(Wd, Wa):
    Wab = jnp.concatenate(
        [Wa[:H], Wa[H:2 * H], jnp.zeros((H, 6), jnp.float32)], axis=1)
    cvec = jnp.broadcast_to((Wd[0, 0] * Wa[2 * H, 0]).reshape(1), (16,))
    return Wab, cvec


def kernel(x, edge_index, edge_d, Wd_o1, W1_o1, W2_o1, Wa_o1, Wd_o2, W1_o2,
           W2_o2, Wa_o2, Wd_d1, W1_d1, W2_d1, Wa_d1, Wd_d2, W1_d2, W2_d2,
           Wa_d2, Wb, bb):
    src = edge_index[0]
    dst = edge_index[1]
    dst2 = dst.reshape(NPLANE, RCHUNK, K)
    ed = edge_d[:, 0]

    def layer_first(h, Wd, W1, W2, Wa):
        Wab, cvec = _layer_weights(Wd, Wa)
        z, zi = _tc_proj_first(h, W1, W2)
        ab = _tc_ab(z, Wab)
        zn = _sc_layer(z, ab[:2], cvec, src, dst2, ed)
        return zi, zn

    def layer_mid(zi_prev, zn_prev, Wd, W1, W2, Wa):
        Wab, cvec = _layer_weights(Wd, Wa)
        z, zi = _tc_proj_mid(zi_prev, zn_prev, W1, W2)
        ab = _tc_ab(z, Wab)
        zn = _sc_layer(z, ab[:2], cvec, src, dst2, ed)
        return zi, zn

    zi_o, zn_o = layer_first(x, Wd_o1, W1_o1, W2_o1, Wa_o1)
    zi_o, zn_o = layer_mid(zi_o, zn_o, Wd_o2, W1_o2, W2_o2, Wa_o2)
    zi_d, zn_d = layer_first(x, Wd_d1, W1_d1, W2_d1, Wa_d1)
    zi_d, zn_d = layer_mid(zi_d, zn_d, Wd_d2, W1_d2, W2_d2, Wa_d2)

    g, hd = _tc_final(zi_o, zn_o, zi_d, zn_d, Wb[0])
    bvec = jnp.broadcast_to(bb.reshape(1), (16,))
    out = _sc_bilinear(g, hd, src, dst, bvec)
    return out.reshape(E, 1)


# speedup vs baseline: 15.4158x; 1.0049x over previous
import dataclasses
import functools

import jax
import jax.numpy as jnp
from jax import lax
from jax.experimental import pallas as pl
from jax.experimental.pallas import tpu as pltpu
from jax.experimental.pallas import tpu_sc as plsc

N = 10000
E = 320000
H = 128

NC = 2
NS = 16
NW = NC * NS
EPW = E // NW
K = 80
NCHUNK = EPW // K
ROUND = 2000
RCHUNK = ROUND // K
NPLANE = E // ROUND
ROWS_PER_TILE = 640

_SC_PARAMS = pltpu.CompilerParams()
if "needs_layout_passes" in pltpu.CompilerParams.__dataclass_fields__:
    _SC_PARAMS = dataclasses.replace(_SC_PARAMS, needs_layout_passes=False)

_MESH = plsc.VectorSubcoreMesh(core_axis_name="c", subcore_axis_name="s")

_PREC = jax.lax.Precision.DEFAULT


def _f32(*shape):
    return jax.ShapeDtypeStruct(shape, jnp.float32)


_BLK = 1000


def _dot(a, b):
    return jax.lax.dot_general(a, b, (((1,), (0,)), ((), ())),
                               precision=_PREC, preferred_element_type=jnp.float32)


def _proj_body(h_ref, w1_ref, w2_ref, z_ref, zi_ref):
    h = h_ref[...]
    z_ref[...] = _dot(h, w1_ref[...])
    zi_ref[...] = _dot(h, w2_ref[...])


def _tc_proj_first(h, W1, W2):
    return pl.pallas_call(
        _proj_body,
        grid=(N // _BLK,),
        in_specs=[
            pl.BlockSpec((_BLK, H), lambda i: (i, 0)),
            pl.BlockSpec((H, H), lambda i: (0, 0)),
            pl.BlockSpec((H, H), lambda i: (0, 0)),
        ],
        out_specs=[
            pl.BlockSpec((_BLK, H), lambda i: (i, 0)),
            pl.BlockSpec((_BLK, H), lambda i: (i, 0)),
        ],
        out_shape=[_f32(N, H), _f32(N, H)],
    )(h, W1, W2)


def _proj_mid_body(zi_prev_ref, zn_ref, w1_ref, w2_ref, z_ref, zi_ref):
    h = jnp.maximum(zi_prev_ref[...] + zn_ref[0] + zn_ref[1], 0.0)
    z_ref[...] = _dot(h, w1_ref[...])
    zi_ref[...] = _dot(h, w2_ref[...])


def _tc_proj_mid(zi_prev, zn_parts, W1, W2):
    return pl.pallas_call(
        _proj_mid_body,
        grid=(N // _BLK,),
        in_specs=[
            pl.BlockSpec((_BLK, H), lambda i: (i, 0)),
            pl.BlockSpec((2, _BLK, H), lambda i: (0, i, 0)),
            pl.BlockSpec((H, H), lambda i: (0, 0)),
            pl.BlockSpec((H, H), lambda i: (0, 0)),
        ],
        out_specs=[
            pl.BlockSpec((_BLK, H), lambda i: (i, 0)),
            pl.BlockSpec((_BLK, H), lambda i: (i, 0)),
        ],
        out_shape=[_f32(N, H), _f32(N, H)],
    )(zi_prev, zn_parts, W1, W2)


def _ab_body(z_ref, wab_ref, ab_ref):
    ab_ref[...] = jax.lax.dot_general(
        wab_ref[...], z_ref[...], (((0,), (1,)), ((), ())),
        precision=_PREC, preferred_element_type=jnp.float32)


def _tc_ab(z, Wab):
    return pl.pallas_call(
        _ab_body,
        in_specs=[
            pl.BlockSpec((N, H), lambda: (0, 0)),
            pl.BlockSpec((H, 8), lambda: (0, 0)),
        ],
        out_specs=pl.BlockSpec((8, N), lambda: (0, 0)),
        out_shape=_f32(8, N),
    )(z, Wab)


def _fin_body(zio_ref, zno_ref, zid_ref, znd_ref, wb_ref, g_ref, hd_ref):
    h_o = jnp.maximum(zio_ref[...] + zno_ref[0] + zno_ref[1], 0.0)
    g_ref[...] = _dot(h_o, wb_ref[...])
    hd_ref[...] = jnp.maximum(zid_ref[...] + znd_ref[0] + znd_ref[1], 0.0)


def _tc_final(zi_o, zn_o, zi_d, zn_d, Wb0):
    return pl.pallas_call(
        _fin_body,
        grid=(N // _BLK,),
        in_specs=[
            pl.BlockSpec((_BLK, H), lambda i: (i, 0)),
            pl.BlockSpec((2, _BLK, H), lambda i: (0, i, 0)),
            pl.BlockSpec((_BLK, H), lambda i: (i, 0)),
            pl.BlockSpec((2, _BLK, H), lambda i: (0, i, 0)),
            pl.BlockSpec((H, H), lambda i: (0, 0)),
        ],
        out_specs=[
            pl.BlockSpec((_BLK, H), lambda i: (i, 0)),
            pl.BlockSpec((_BLK, H), lambda i: (i, 0)),
        ],
        out_shape=[_f32(N, H), _f32(N, H)],
    )(zi_o, zn_o, zi_d, zn_d, Wb0)


def _leaky(x):
    return jnp.where(x >= 0.0, x, 0.01 * x)


@functools.partial(
    pl.kernel,
    out_type=_f32(NC, N, H),
    mesh=_MESH,
    compiler_params=_SC_PARAMS,
    scratch_types=[
        pltpu.VMEM((N,), jnp.float32),
        pltpu.VMEM((N,), jnp.float32),
        pltpu.VMEM((ROUND,), jnp.float32),
        pltpu.VMEM((ROUND,), jnp.int32),
        pltpu.VMEM((RCHUNK, K), jnp.int32),
        pltpu.VMEM((ROUND,), jnp.float32),
        pltpu.VMEM((K,), jnp.float32),
        pltpu.VMEM((K, H), jnp.float32),
        pltpu.VMEM((K,), jnp.float32),
        pltpu.VMEM((16,), jnp.float32),
        pltpu.VMEM_SHARED((N,), jnp.float32),
        pltpu.VMEM_SHARED((N, H), jnp.float32),
        pltpu.SemaphoreType.DMA,
    ],
)
def _sc_layer(z_hbm, ab_hbm, cvec_hbm, src_hbm, dst2_hbm, ed_hbm, out_hbm,
              a_v, b_v, ee_v, src_v, dst_v, ed_v, den_v, rows_v, al_v, c_v,
              den_sh, zn_sh, sem):
    c = lax.axis_index("c")
    s = lax.axis_index("s")
    w = c * NS + s

    pltpu.sync_copy(ab_hbm.at[0], a_v)
    pltpu.sync_copy(ab_hbm.at[1], b_v)
    pltpu.sync_copy(cvec_hbm, c_v)
    coeff = c_v[...]

    zero16 = jnp.zeros((16,), jnp.float32)

    @pl.loop(0, K)
    def _(r):
        for q in range(H // 16):
            rows_v[r, pl.ds(q * 16, 16)] = zero16

    nrows = jnp.where(s == NS - 1, N - (NS - 1) * ROWS_PER_TILE, ROWS_PER_TILE)
    base_row = s * ROWS_PER_TILE

    @pl.loop(0, nrows, step=K)
    def _(r0):
        pltpu.sync_copy(rows_v, zn_sh.at[pl.ds(base_row + r0, K)])

    @pl.when(s == 0)
    def _():
        @pl.loop(0, ROUND, step=16)
        def _(i):
            ee_v[pl.ds(i, 16)] = zero16

        @pl.loop(0, N, step=ROUND)
        def _(i):
            pltpu.sync_copy(ee_v, den_sh.at[pl.ds(i, ROUND)])

    plsc.subcore_barrier()

    @pl.loop(0, 2 * EPW // ROUND)
    def _(rr):
        base = s * (2 * EPW) + rr * ROUND
        plane = s * (2 * EPW // ROUND) + rr
        pltpu.sync_copy(src_hbm.at[pl.ds(base, ROUND)], src_v)
        pltpu.sync_copy(dst2_hbm.at[plane], dst_v)
        pltpu.sync_copy(ed_hbm.at[pl.ds(base, ROUND)], ed_v)

        @pl.loop(0, RCHUNK)
        def _(j):
            for g in range(K // 16):
                o16 = j * K + g * 16
                s16 = src_v[pl.ds(o16, 16)]
                d16 = dst_v[j, pl.ds(g * 16, 16)]
                av = plsc.load_gather(a_v, [s16])
                bv = plsc.load_gather(b_v, [d16])
                ed16 = ed_v[pl.ds(o16, 16)]
                ee_v[pl.ds(o16, 16)] = jnp.exp(_leaky(av + bv + coeff * ed16))

        @pl.loop(0, RCHUNK)
        def _(j):
            pltpu.async_copy(ee_v.at[pl.ds(j * K, K)], den_sh.at[dst_v.at[j]],
                             sem, add=True)

        @pl.loop(0, RCHUNK)
        def _(j):
            pltpu.make_async_copy(ee_v.at[pl.ds(0, K)],
                                  den_sh.at[dst_v.at[0]], sem).wait()

    plsc.subcore_barrier()

    @pl.loop(0, EPW // ROUND)
    def _(rr):
        base = w * EPW + rr * ROUND
        plane = w * (EPW // ROUND) + rr
        pltpu.sync_copy(src_hbm.at[pl.ds(base, ROUND)], src_v)
        pltpu.sync_copy(dst2_hbm.at[plane], dst_v)
        pltpu.sync_copy(ed_hbm.at[pl.ds(base, ROUND)], ed_v)

        @pl.loop(0, RCHUNK)
        def _(j):
            pltpu.sync_copy(z_hbm.at[src_v.at[pl.ds(j * K, K)]], rows_v)
            pltpu.sync_copy(den_sh.at[dst_v.at[j]], den_v)
            for g in range(K // 16):
                o16 = j * K + g * 16
                s16 = src_v[pl.ds(o16, 16)]
                d16 = dst_v[j, pl.ds(g * 16, 16)]
                av = plsc.load_gather(a_v, [s16])
                bv = plsc.load_gather(b_v, [d16])
                ed16 = ed_v[pl.ds(o16, 16)]
                ee16 = jnp.exp(_leaky(av + bv + coeff * ed16))
                dv = den_v[pl.ds(g * 16, 16)]
                al_v[pl.ds(g * 16, 16)] = ee16 / (dv + 1e-9)

            @pl.loop(0, K)
            def _(r):
                sc = plsc.load_gather(al_v, [jnp.broadcast_to(r, (16,))])
                for q in range(H // 16):
                    rows_v[r, pl.ds(q * 16, 16)] = (
                        rows_v[r, pl.ds(q * 16, 16)] * sc)

            pltpu.sync_copy(rows_v, zn_sh.at[dst_v.at[j]], add=True)

    plsc.subcore_barrier()

    @pl.loop(0, nrows, step=K)
    def _(r0):
        pltpu.sync_copy(zn_sh.at[pl.ds(base_row + r0, K)],
                        out_hbm.at[c, pl.ds(base_row + r0, K)])


@functools.partial(
    pl.kernel,
    out_type=_f32(E),
    mesh=_MESH,
    compiler_params=_SC_PARAMS,
    scratch_types=[
        pltpu.VMEM((EPW,), jnp.int32),
        pltpu.VMEM((EPW,), jnp.int32),
        pltpu.VMEM((K, H), jnp.float32),
        pltpu.VMEM((K, H), jnp.float32),
        pltpu.VMEM((K, H), jnp.float32),
        pltpu.VMEM((K, H), jnp.float32),
        pltpu.VMEM((EPW,), jnp.float32),
        pltpu.VMEM((16,), jnp.float32),
        pltpu.VMEM((16, 16), jnp.float32),
        pltpu.SemaphoreType.DMA,
        pltpu.SemaphoreType.DMA,
    ],
)
def _sc_bilinear(g_hbm, hd_hbm, src_hbm, dst_hbm, bvec_hbm, out_hbm,
                 src_v, dst_v, gr0, hr0, gr1, hr1, o_v, b_v, t_v,
                 semg0, semg1):
    c = lax.axis_index("c")
    s = lax.axis_index("s")
    w = c * NS + s
    base = w * EPW

    pltpu.sync_copy(bvec_hbm, b_v)
    bias = b_v[...]
    pltpu.sync_copy(src_hbm.at[pl.ds(base, EPW)], src_v)
    pltpu.sync_copy(dst_hbm.at[pl.ds(base, EPW)], dst_v)

    lane = lax.iota(jnp.int32, 16)

    def b_start(j, gr, hr, semg):
        pltpu.async_copy(g_hbm.at[src_v.at[pl.ds(j * K, K)]], gr, semg)
        pltpu.async_copy(hd_hbm.at[dst_v.at[pl.ds(j * K, K)]], hr, semg)

    def b_wait(j, gr, hr, semg):
        pltpu.make_async_copy(g_hbm.at[src_v.at[pl.ds(j * K, K)]], gr,
                              semg).wait()
        pltpu.make_async_copy(hd_hbm.at[dst_v.at[pl.ds(j * K, K)]], hr,
                              semg).wait()

    def b_compute(j, gr, hr):
        @pl.loop(0, K, step=16)
        def _(r0):
            @pl.loop(0, 16)
            def _(rr):
                r = r0 + rr
                acc = gr[r, pl.ds(0, 16)] * hr[r, pl.ds(0, 16)]
                for q in range(1, H // 16):
                    acc = acc + gr[r, pl.ds(q * 16, 16)] * hr[r, pl.ds(q * 16, 16)]
                t_v[rr, pl.ds(0, 16)] = acc

            tot = bias
            for ell in range(16):
                tot = tot + plsc.load_gather(
                    t_v, [lane, jnp.broadcast_to(ell, (16,))])
            o_v[pl.ds(j * K + r0, 16)] = tot

    b_start(0, gr0, hr0, semg0)

    @pl.loop(0, NCHUNK - 1, step=2)
    def _(j):
        b_wait(j, gr0, hr0, semg0)
        b_start(j + 1, gr1, hr1, semg1)
        b_compute(j, gr0, hr0)
        b_wait(j + 1, gr1, hr1, semg1)
        b_start(j + 2, gr0, hr0, semg0)
        b_compute(j + 1, gr1, hr1)

    b_wait(NCHUNK - 1, gr0, hr0, semg0)
    b_compute(NCHUNK - 1, gr0, hr0)

    pltpu.sync_copy(o_v, out_hbm.at[pl.ds(base, EPW)])


def _layer_weights(Wd, Wa):
    Wab = jnp.concatenate(
        [Wa[:H], Wa[H:2 * H], jnp.zeros((H, 6), jnp.float32)], axis=1)
    cvec = jnp.broadcast_to((Wd[0, 0] * Wa[2 * H, 0]).reshape(1), (16,))
    return Wab, cvec


def kernel(x, edge_index, edge_d, Wd_o1, W1_o1, W2_o1, Wa_o1, Wd_o2, W1_o2,
           W2_o2, Wa_o2, Wd_d1, W1_d1, W2_d1, Wa_d1, Wd_d2, W1_d2, W2_d2,
           Wa_d2, Wb, bb):
    src = edge_index[0]
    dst = edge_index[1]
    dst2 = dst.reshape(NPLANE, RCHUNK, K)
    ed = edge_d[:, 0]

    def layer_first(h, Wd, W1, W2, Wa):
        Wab, cvec = _layer_weights(Wd, Wa)
        z, zi = _tc_proj_first(h, W1, W2)
        ab = _tc_ab(z, Wab)
        zn = _sc_layer(z, ab[:2], cvec, src, dst2, ed)
        return zi, zn

    def layer_mid(zi_prev, zn_prev, Wd, W1, W2, Wa):
        Wab, cvec = _layer_weights(Wd, Wa)
        z, zi = _tc_proj_mid(zi_prev, zn_prev, W1, W2)
        ab = _tc_ab(z, Wab)
        zn = _sc_layer(z, ab[:2], cvec, src, dst2, ed)
        return zi, zn

    zi_o, zn_o = layer_first(x, Wd_o1, W1_o1, W2_o1, Wa_o1)
    zi_o, zn_o = layer_mid(zi_o, zn_o, Wd_o2, W1_o2, W2_o2, Wa_o2)
    zi_d, zn_d = layer_first(x, Wd_d1, W1_d1, W2_d1, Wa_d1)
    zi_d, zn_d = layer_mid(zi_d, zn_d, Wd_d2, W1_d2, W2_d2, Wa_d2)

    g, hd = _tc_final(zi_o, zn_o, zi_d, zn_d, Wb[0])
    bvec = jnp.broadcast_to(bb.reshape(1), (16,))
    out = _sc_bilinear(g, hd, src, dst, bvec)
    return out.reshape(E, 1)
